# baseline (speedup 1.0000x reference)
import jax
import jax.numpy as jnp
from jax import lax
from jax.experimental import pallas as pl
from jax.experimental.pallas import tpu as pltpu
from jax.experimental.pallas import tpu_sc as plsc

NC = 2
NS = 16
NW = NC * NS
LANES = 16
K = 4

N_NODES = 100000
N_LINKS = 199350
NLPAD = 199360
ZPOS = 2 * NLPAD
TABLE = ZPOS + LANES
N_PER_W = 3136
N_TAIL = N_NODES - (NW - 1) * N_PER_W
IDX_PER_W = K * N_PER_W
CHUNKS = N_PER_W // LANES
TAIL_CHUNKS = N_TAIL // LANES
STAGE = 12464
STAGE_TAIL = N_LINKS - (NS - 1) * STAGE
STAGE_TAIL_UP = 12400


def _sc_body(array_hbm, encT_hbm, status_hbm, area_hbm, out_hbm,
             table_sp, idx_v, gath_v, status_v, area_v, out_v, sem, sem2):
    cid = lax.axis_index("c")
    sid = lax.axis_index("s")
    wid = sid * NC + cid
    is_tail = wid == NW - 1
    nbase = pl.multiple_of(wid * N_PER_W, 8)
    stage_off = pl.multiple_of(sid * STAGE, 8)

    def stage_pair(cnt):
        return (array_hbm.at[pl.ds(stage_off, cnt)], gath_v.at[pl.ds(0, cnt)])

    def chunk_pairs(cnt):
        prs = []
        for j in range(K):
            prs.append((encT_hbm.at[pl.ds(j * N_NODES + nbase, cnt)],
                        idx_v.at[pl.ds(j * N_PER_W, cnt)]))
        prs.append((status_hbm.at[pl.ds(nbase, cnt)],
                    status_v.at[pl.ds(0, cnt)]))
        prs.append((area_hbm.at[pl.ds(nbase, cnt)],
                    area_v.at[pl.ds(0, cnt)]))
        return prs

    @pl.when(sid < NS - 1)
    def _():
        s, d = stage_pair(STAGE)
        pltpu.async_copy(s, d, sem2)

    @pl.when(sid == NS - 1)
    def _():
        s, d = stage_pair(STAGE_TAIL)
        pltpu.async_copy(s, d, sem2)

    @pl.when(jnp.logical_not(is_tail))
    def _():
        for s, d in chunk_pairs(N_PER_W):
            pltpu.async_copy(s, d, sem)

    @pl.when(is_tail)
    def _():
        for s, d in chunk_pairs(N_TAIL):
            pltpu.async_copy(s, d, sem)

        def zfill(i, carry):
            j, c = i // (CHUNKS - TAIL_CHUNKS), i % (CHUNKS - TAIL_CHUNKS)
            idx_v[pl.ds(j * N_PER_W + N_TAIL + c * LANES, LANES)] = (
                jnp.full((LANES,), ZPOS, jnp.int32))
            return carry
        lax.fori_loop(0, K * (CHUNKS - TAIL_CHUNKS), zfill, 0)

    def publish(cnt, neg_cnt):
        pltpu.make_async_copy(*stage_pair(cnt), sem2).wait()
        pltpu.sync_copy(gath_v.at[pl.ds(0, cnt)],
                        table_sp.at[pl.ds(NLPAD + stage_off, cnt)])

        def negate(i, carry):
            v = gath_v[pl.ds(i * LANES, LANES)]
            gath_v[pl.ds(i * LANES, LANES)] = -v
            return carry
        lax.fori_loop(0, neg_cnt // LANES, negate, 0)
        pltpu.sync_copy(gath_v.at[pl.ds(0, neg_cnt)],
                        table_sp.at[pl.ds(stage_off, neg_cnt)])

    @pl.when(sid < NS - 1)
    def _():
        publish(STAGE, STAGE)

    @pl.when(sid == NS - 1)
    def _():
        publish(STAGE_TAIL, STAGE_TAIL_UP)

    @pl.when(sid == 0)
    def _():
        out_v[pl.ds(0, LANES)] = jnp.zeros((LANES,), jnp.float32)
        pltpu.sync_copy(out_v.at[pl.ds(0, LANES)],
                        table_sp.at[pl.ds(ZPOS, LANES)])

    @pl.when(jnp.logical_not(is_tail))
    def _():
        for s, d in chunk_pairs(N_PER_W):
            pltpu.make_async_copy(s, d, sem).wait()

    @pl.when(is_tail)
    def _():
        for s, d in chunk_pairs(N_TAIL):
            pltpu.make_async_copy(s, d, sem).wait()

    plsc.subcore_barrier()

    pltpu.async_copy(table_sp.at[idx_v], gath_v, sem).wait()

    def chunk(c, carry):
        off = c * LANES
        acc = gath_v[pl.ds(off, LANES)]
        for j in range(1, K):
            acc = acc + gath_v[pl.ds(j * N_PER_W + off, LANES)]
        st = status_v[pl.ds(off, LANES)]
        ar = area_v[pl.ds(off, LANES)]
        out_v[pl.ds(off, LANES)] = jnp.where(st == 0, acc / ar, 0.0)
        return carry

    lax.fori_loop(0, CHUNKS, chunk, 0)

    @pl.when(jnp.logical_not(is_tail))
    def _():
        pltpu.sync_copy(out_v, out_hbm.at[pl.ds(nbase, N_PER_W)])

    @pl.when(is_tail)
    def _():
        pltpu.sync_copy(out_v.at[pl.ds(0, N_TAIL)],
                        out_hbm.at[pl.ds(nbase, N_TAIL)])


@jax.jit
def _flux_div_sc(array, enc_T, status, area):
    mesh = plsc.VectorSubcoreMesh(core_axis_name="c", subcore_axis_name="s")
    run = pl.kernel(
        _sc_body,
        out_type=jax.ShapeDtypeStruct((N_NODES,), jnp.float32),
        mesh=mesh,
        scratch_types=[
            pltpu.VMEM_SHARED((TABLE,), jnp.float32),
            pltpu.VMEM((IDX_PER_W,), jnp.int32),
            pltpu.VMEM((IDX_PER_W,), jnp.float32),
            pltpu.VMEM((N_PER_W,), jnp.int32),
            pltpu.VMEM((N_PER_W,), jnp.float32),
            pltpu.VMEM((N_PER_W,), jnp.float32),
            pltpu.SemaphoreType.DMA,
            pltpu.SemaphoreType.DMA,
        ],
        compiler_params=pltpu.CompilerParams(needs_layout_passes=False),
    )
    return run(array, enc_T, status, area)


def kernel(array, cell_area_at_node, links_at_node, link_dirs_at_node, status_at_node):
    enc = jnp.where(
        link_dirs_at_node == 0,
        ZPOS,
        links_at_node + jnp.where(link_dirs_at_node > 0, NLPAD, 0),
    ).astype(jnp.int32)
    enc_T = jnp.swapaxes(enc, 0, 1).reshape(K * N_NODES)
    return _flux_div_sc(array, enc_T, status_at_node, cell_area_at_node)


# speedup vs baseline: 2.2264x; 2.2264x over previous
import jax
import jax.numpy as jnp
from jax import lax
from jax.experimental import pallas as pl
from jax.experimental.pallas import tpu as pltpu
from jax.experimental.pallas import tpu_sc as plsc

NC = 2
NS = 16
NW = NC * NS
LANES = 16
K = 4

N_NODES = 100000
N_LINKS = 199350
NLPAD = 199360
ZPOS = 2 * NLPAD
ZBLK = 4096
ZPER = ZBLK // NS
TABLE = ZPOS + ZBLK
N_PER_W = 3136
N_TAIL = N_NODES - (NW - 1) * N_PER_W
IDX_PER_W = K * N_PER_W
CHUNKS = N_PER_W // LANES
TAIL_CHUNKS = N_TAIL // LANES
STAGE = 12464
STAGE_TAIL = N_LINKS - (NS - 1) * STAGE
STAGE_TAIL_UP = 12400


def _sc_body(array_hbm, encT_hbm, status_hbm, area_hbm, out_hbm,
             table_sp, idx_v, gath_v, neg_v, status_v, area_v, out_v,
             sem, sem2):
    cid = lax.axis_index("c")
    sid = lax.axis_index("s")
    wid = sid * NC + cid
    is_tail = wid == NW - 1
    nbase = pl.multiple_of(wid * N_PER_W, 8)
    stage_off = pl.multiple_of(sid * STAGE, 8)

    def stage_pair(cnt):
        return (array_hbm.at[pl.ds(stage_off, cnt)], gath_v.at[pl.ds(0, cnt)])

    def chunk_pairs(cnt):
        prs = []
        for j in range(K):
            prs.append((encT_hbm.at[pl.ds(j * N_NODES + nbase, cnt)],
                        idx_v.at[pl.ds(j * N_PER_W, cnt)]))
        prs.append((status_hbm.at[pl.ds(nbase, cnt)],
                    status_v.at[pl.ds(0, cnt)]))
        prs.append((area_hbm.at[pl.ds(nbase, cnt)],
                    area_v.at[pl.ds(0, cnt)]))
        return prs

    @pl.when(sid < NS - 1)
    def _():
        s, d = stage_pair(STAGE)
        pltpu.async_copy(s, d, sem2)

    @pl.when(sid == NS - 1)
    def _():
        s, d = stage_pair(STAGE_TAIL)
        pltpu.async_copy(s, d, sem2)

    @pl.when(jnp.logical_not(is_tail))
    def _():
        for s, d in chunk_pairs(N_PER_W):
            pltpu.async_copy(s, d, sem)

    @pl.when(is_tail)
    def _():
        for s, d in chunk_pairs(N_TAIL):
            pltpu.async_copy(s, d, sem)

        @plsc.parallel_loop(0, K * (CHUNKS - TAIL_CHUNKS), unroll=4)
        def _(i):
            j, c = i // (CHUNKS - TAIL_CHUNKS), i % (CHUNKS - TAIL_CHUNKS)
            idx_v[pl.ds(j * N_PER_W + N_TAIL + c * LANES, LANES)] = (
                jnp.full((LANES,), ZPOS, jnp.int32))

    @plsc.parallel_loop(0, ZPER // LANES, unroll=4)
    def _(i):
        neg_v[pl.ds(i * LANES, LANES)] = jnp.zeros((LANES,), jnp.float32)

    pltpu.sync_copy(neg_v.at[pl.ds(0, ZPER)],
                    table_sp.at[pl.ds(ZPOS + sid * ZPER, ZPER)])

    def publish(cnt, neg_cnt):
        pltpu.make_async_copy(*stage_pair(cnt), sem2).wait()
        pos = pltpu.async_copy(
            gath_v.at[pl.ds(0, cnt)],
            table_sp.at[pl.ds(NLPAD + stage_off, cnt)], sem2)

        @plsc.parallel_loop(0, neg_cnt // LANES, unroll=4)
        def _(i):
            neg_v[pl.ds(i * LANES, LANES)] = -gath_v[pl.ds(i * LANES, LANES)]

        pltpu.sync_copy(neg_v.at[pl.ds(0, neg_cnt)],
                        table_sp.at[pl.ds(stage_off, neg_cnt)])
        pos.wait()

    @pl.when(sid < NS - 1)
    def _():
        publish(STAGE, STAGE)

    @pl.when(sid == NS - 1)
    def _():
        publish(STAGE_TAIL, STAGE_TAIL_UP)

    @pl.when(jnp.logical_not(is_tail))
    def _():
        for s, d in chunk_pairs(N_PER_W):
            pltpu.make_async_copy(s, d, sem).wait()

    @pl.when(is_tail)
    def _():
        for s, d in chunk_pairs(N_TAIL):
            pltpu.make_async_copy(s, d, sem).wait()

    plsc.subcore_barrier()

    pltpu.async_copy(table_sp.at[idx_v], gath_v, sem).wait()

    @plsc.parallel_loop(0, CHUNKS, unroll=4)
    def _(c):
        off = c * LANES
        acc = gath_v[pl.ds(off, LANES)]
        for j in range(1, K):
            acc = acc + gath_v[pl.ds(j * N_PER_W + off, LANES)]
        st = status_v[pl.ds(off, LANES)]
        ar = area_v[pl.ds(off, LANES)]
        out_v[pl.ds(off, LANES)] = jnp.where(st == 0, acc / ar, 0.0)

    @pl.when(jnp.logical_not(is_tail))
    def _():
        pltpu.sync_copy(out_v, out_hbm.at[pl.ds(nbase, N_PER_W)])

    @pl.when(is_tail)
    def _():
        pltpu.sync_copy(out_v.at[pl.ds(0, N_TAIL)],
                        out_hbm.at[pl.ds(nbase, N_TAIL)])


@jax.jit
def _flux_div_sc(array, enc_T, status, area):
    mesh = plsc.VectorSubcoreMesh(core_axis_name="c", subcore_axis_name="s")
    run = pl.kernel(
        _sc_body,
        out_type=jax.ShapeDtypeStruct((N_NODES,), jnp.float32),
        mesh=mesh,
        scratch_types=[
            pltpu.VMEM_SHARED((TABLE,), jnp.float32),
            pltpu.VMEM((IDX_PER_W,), jnp.int32),
            pltpu.VMEM((IDX_PER_W,), jnp.float32),
            pltpu.VMEM((IDX_PER_W,), jnp.float32),
            pltpu.VMEM((N_PER_W,), jnp.int32),
            pltpu.VMEM((N_PER_W,), jnp.float32),
            pltpu.VMEM((N_PER_W,), jnp.float32),
            pltpu.SemaphoreType.DMA,
            pltpu.SemaphoreType.DMA,
        ],
        compiler_params=pltpu.CompilerParams(needs_layout_passes=False),
    )
    return run(array, enc_T, status, area)


def kernel(array, cell_area_at_node, links_at_node, link_dirs_at_node, status_at_node):
    enc = jnp.where(
        link_dirs_at_node == 0,
        ZPOS + (links_at_node & (ZBLK - 1)),
        links_at_node + jnp.where(link_dirs_at_node > 0, NLPAD, 0),
    ).astype(jnp.int32)
    enc_T = jnp.swapaxes(enc, 0, 1).reshape(K * N_NODES)
    return _flux_div_sc(array, enc_T, status_at_node, cell_area_at_node)


# speedup vs baseline: 2.3191x; 1.0416x over previous
import jax
import jax.numpy as jnp
from jax import lax
from jax.experimental import pallas as pl
from jax.experimental.pallas import tpu as pltpu
from jax.experimental.pallas import tpu_sc as plsc

NC = 2
NS = 16
NW = NC * NS
LANES = 16
K = 4

N_NODES = 100000
N_LINKS = 199350
NLPAD = 199360
ZPOS = 2 * NLPAD
ZBLK = 4096
ZPER = ZBLK // NS
TABLE = ZPOS + ZBLK
N_W0 = 3312
N_W1 = 2944
N_TAIL = N_NODES - NS * N_W0 - (NS - 1) * N_W1
SC1_BASE = NS * N_W0
IDX_PER_W = K * N_W0
CHUNKS0 = N_W0 // LANES
CHUNKS1 = N_W1 // LANES
TAIL_CHUNKS = N_TAIL // LANES
STAGE = 12464
STAGE_TAIL = N_LINKS - (NS - 1) * STAGE
STAGE_TAIL_UP = 12400


def _sc_body(array_hbm, encT_hbm, status_hbm, area_hbm, out_hbm,
             table_sp, idx_v, gath_v, neg_v, status_v, area_v, out_v,
             sem, sem2):
    cid = lax.axis_index("c")
    sid = lax.axis_index("s")
    is_c0 = cid == 0
    is_tail = jnp.logical_and(cid == 1, sid == NS - 1)
    is_mid1 = jnp.logical_and(cid == 1, sid < NS - 1)
    base0 = pl.multiple_of(sid * N_W0, 8)
    base1 = pl.multiple_of(SC1_BASE + sid * N_W1, 8)
    stage_off = pl.multiple_of(sid * STAGE, 8)

    def stage_pair(cnt):
        return (array_hbm.at[pl.ds(stage_off, cnt)], gath_v.at[pl.ds(0, cnt)])

    def chunk_pairs(base, cnt, stride):
        prs = []
        for j in range(K):
            prs.append((encT_hbm.at[pl.ds(j * N_NODES + base, cnt)],
                        idx_v.at[pl.ds(j * stride, cnt)]))
        prs.append((status_hbm.at[pl.ds(base, cnt)],
                    status_v.at[pl.ds(0, cnt)]))
        prs.append((area_hbm.at[pl.ds(base, cnt)],
                    area_v.at[pl.ds(0, cnt)]))
        return prs

    @pl.when(sid < NS - 1)
    def _():
        s, d = stage_pair(STAGE)
        pltpu.async_copy(s, d, sem2)

    @pl.when(sid == NS - 1)
    def _():
        s, d = stage_pair(STAGE_TAIL)
        pltpu.async_copy(s, d, sem2)

    @pl.when(is_c0)
    def _():
        for s, d in chunk_pairs(base0, N_W0, N_W0):
            pltpu.async_copy(s, d, sem)

    @pl.when(is_mid1)
    def _():
        for s, d in chunk_pairs(base1, N_W1, N_W1):
            pltpu.async_copy(s, d, sem)

    @pl.when(is_tail)
    def _():
        for s, d in chunk_pairs(base1, N_TAIL, N_W1):
            pltpu.async_copy(s, d, sem)

        @plsc.parallel_loop(0, K * (CHUNKS1 - TAIL_CHUNKS), unroll=2)
        def _(i):
            j, c = i // (CHUNKS1 - TAIL_CHUNKS), i % (CHUNKS1 - TAIL_CHUNKS)
            idx_v[pl.ds(j * N_W1 + N_TAIL + c * LANES, LANES)] = (
                jnp.full((LANES,), ZPOS, jnp.int32))

    @plsc.parallel_loop(0, ZPER // LANES, unroll=4)
    def _(i):
        neg_v[pl.ds(i * LANES, LANES)] = jnp.zeros((LANES,), jnp.float32)

    pltpu.sync_copy(neg_v.at[pl.ds(0, ZPER)],
                    table_sp.at[pl.ds(ZPOS + sid * ZPER, ZPER)])

    def publish(cnt, neg_cnt):
        pltpu.make_async_copy(*stage_pair(cnt), sem2).wait()
        pos = pltpu.async_copy(
            gath_v.at[pl.ds(0, cnt)],
            table_sp.at[pl.ds(NLPAD + stage_off, cnt)], sem2)

        @plsc.parallel_loop(0, neg_cnt // LANES, unroll=4)
        def _(i):
            neg_v[pl.ds(i * LANES, LANES)] = -gath_v[pl.ds(i * LANES, LANES)]

        pltpu.sync_copy(neg_v.at[pl.ds(0, neg_cnt)],
                        table_sp.at[pl.ds(stage_off, neg_cnt)])
        pos.wait()

    @pl.when(sid < NS - 1)
    def _():
        publish(STAGE, STAGE)

    @pl.when(sid == NS - 1)
    def _():
        publish(STAGE_TAIL, STAGE_TAIL_UP)

    @pl.when(is_c0)
    def _():
        for s, d in chunk_pairs(base0, N_W0, N_W0):
            pltpu.make_async_copy(s, d, sem).wait()

    @pl.when(is_mid1)
    def _():
        for s, d in chunk_pairs(base1, N_W1, N_W1):
            pltpu.make_async_copy(s, d, sem).wait()

    @pl.when(is_tail)
    def _():
        for s, d in chunk_pairs(base1, N_TAIL, N_W1):
            pltpu.make_async_copy(s, d, sem).wait()

    plsc.subcore_barrier()

    def tail_fn(stride, chunks):
        def fn():
            pltpu.async_copy(table_sp.at[idx_v.at[pl.ds(0, K * stride)]],
                             gath_v.at[pl.ds(0, K * stride)], sem).wait()

            @plsc.parallel_loop(0, chunks, unroll=4)
            def _(c):
                off = c * LANES
                acc = gath_v[pl.ds(off, LANES)]
                for j in range(1, K):
                    acc = acc + gath_v[pl.ds(j * stride + off, LANES)]
                st = status_v[pl.ds(off, LANES)]
                ar = area_v[pl.ds(off, LANES)]
                out_v[pl.ds(off, LANES)] = jnp.where(st == 0, acc / ar, 0.0)
        return fn

    pl.when(is_c0)(tail_fn(N_W0, CHUNKS0))
    pl.when(jnp.logical_not(is_c0))(tail_fn(N_W1, CHUNKS1))

    @pl.when(is_c0)
    def _():
        pltpu.sync_copy(out_v.at[pl.ds(0, N_W0)],
                        out_hbm.at[pl.ds(base0, N_W0)])

    @pl.when(is_mid1)
    def _():
        pltpu.sync_copy(out_v.at[pl.ds(0, N_W1)],
                        out_hbm.at[pl.ds(base1, N_W1)])

    @pl.when(is_tail)
    def _():
        pltpu.sync_copy(out_v.at[pl.ds(0, N_TAIL)],
                        out_hbm.at[pl.ds(base1, N_TAIL)])


@jax.jit
def _flux_div_sc(array, enc_T, status, area):
    mesh = plsc.VectorSubcoreMesh(core_axis_name="c", subcore_axis_name="s")
    run = pl.kernel(
        _sc_body,
        out_type=jax.ShapeDtypeStruct((N_NODES,), jnp.float32),
        mesh=mesh,
        scratch_types=[
            pltpu.VMEM_SHARED((TABLE,), jnp.float32),
            pltpu.VMEM((IDX_PER_W,), jnp.int32),
            pltpu.VMEM((IDX_PER_W,), jnp.float32),
            pltpu.VMEM((IDX_PER_W,), jnp.float32),
            pltpu.VMEM((N_W0,), jnp.int32),
            pltpu.VMEM((N_W0,), jnp.float32),
            pltpu.VMEM((N_W0,), jnp.float32),
            pltpu.SemaphoreType.DMA,
            pltpu.SemaphoreType.DMA,
        ],
        compiler_params=pltpu.CompilerParams(needs_layout_passes=False),
    )
    return run(array, enc_T, status, area)


def kernel(array, cell_area_at_node, links_at_node, link_dirs_at_node, status_at_node):
    enc = jnp.where(
        link_dirs_at_node == 0,
        ZPOS + (links_at_node & (ZBLK - 1)),
        links_at_node + jnp.where(link_dirs_at_node > 0, NLPAD, 0),
    ).astype(jnp.int32)
    enc_T = jnp.swapaxes(enc, 0, 1).reshape(K * N_NODES)
    return _flux_div_sc(array, enc_T, status_at_node, cell_area_at_node)
